# Initial kernel scaffold; baseline (speedup 1.0000x reference)
import jax
import jax.numpy as jnp
from jax.experimental import pallas as pl


def kernel(x, edge_index, in_emb_weight, out_emb_weight):
    raise NotImplementedError("write your pallas kernel here")


# speedup vs baseline: 792.9864x; 792.9864x over previous
import functools

import jax
import jax.numpy as jnp
from jax import lax
from jax.experimental import pallas as pl
from jax.experimental.pallas import tpu as pltpu
from jax.experimental.pallas import tpu_sc as plsc

N_NODES = 10000
N_EDGES = 320000
EMB_DIM = 128
MAX_DEG = 256

NC = 2
NS = 16
NW = NC * NS
EDGES_PER_W = N_EDGES // NW
BINS = 10240
BINS_PER_TILE = BINS // NS
NODE_BLK = 512
TC_GRID = (N_NODES + NODE_BLK - 1) // NODE_BLK


def _i(v):
    return jnp.int32(v)


def _sc_degree_body(src_hbm, dst_hbm, out_hbm,
                    src_v, dst_v, hin_v, hout_v, shared, red_v, res_v, sem):
    c = lax.axis_index("c")
    s = lax.axis_index("s")
    w = c * _i(NS) + s
    base = w * _i(EDGES_PER_W)

    cp_s = pltpu.async_copy(src_hbm.at[pl.ds(base, EDGES_PER_W)], src_v, sem)
    cp_d = pltpu.async_copy(dst_hbm.at[pl.ds(base, EDGES_PER_W)], dst_v, sem)

    zeros = jnp.zeros((16,), jnp.int32)

    def zbody(i, carry):
        o = i * _i(16)
        hin_v[pl.ds(o, 16)] = zeros
        hout_v[pl.ds(o, 16)] = zeros
        return carry

    lax.fori_loop(_i(0), _i(BINS // 16), zbody, None)

    cp_s.wait()
    cp_d.wait()

    ones = jnp.ones((16,), jnp.int32)

    def sbody(i, carry):
        o = i * _i(16)
        sv = src_v[pl.ds(o, 16)]
        dv = dst_v[pl.ds(o, 16)]
        m = sv != dv
        plsc.addupdate_scatter(hout_v, [sv], ones, mask=m)
        plsc.addupdate_scatter(hin_v, [dv], ones, mask=m)
        return carry

    lax.fori_loop(_i(0), _i(EDGES_PER_W // 16), sbody, None)

    pltpu.sync_copy(hin_v, shared.at[s, _i(0)])
    pltpu.sync_copy(hout_v, shared.at[s, _i(1)])
    plsc.subcore_barrier()

    col = s * _i(BINS_PER_TILE)
    for io in range(2):
        for t in range(NS):
            pltpu.sync_copy(shared.at[_i(t), _i(io), pl.ds(col, BINS_PER_TILE)],
                            red_v.at[_i(t)])

        def rbody(j, carry):
            o = j * _i(16)
            acc = red_v[_i(0), pl.ds(o, 16)]
            for t in range(1, NS):
                acc = acc + red_v[_i(t), pl.ds(o, 16)]
            res_v[pl.ds(o, 16)] = acc
            return carry

        lax.fori_loop(_i(0), _i(BINS_PER_TILE // 16), rbody, None)
        pltpu.sync_copy(res_v, out_hbm.at[c * _i(2) + _i(io), pl.ds(col, BINS_PER_TILE)])


@functools.cache
def _get_sc_degree():
    return functools.partial(
        pl.kernel,
        out_type=jax.ShapeDtypeStruct((8, BINS), jnp.int32),
        mesh=plsc.VectorSubcoreMesh(core_axis_name="c", subcore_axis_name="s"),
        compiler_params=pltpu.CompilerParams(needs_layout_passes=False),
        scratch_types=[
            pltpu.VMEM((EDGES_PER_W,), jnp.int32),
            pltpu.VMEM((EDGES_PER_W,), jnp.int32),
            pltpu.VMEM((BINS,), jnp.int32),
            pltpu.VMEM((BINS,), jnp.int32),
            pltpu.VMEM_SHARED((NS, 2, BINS), jnp.int32),
            pltpu.VMEM((NS, BINS_PER_TILE), jnp.int32),
            pltpu.VMEM((BINS_PER_TILE,), jnp.int32),
            pltpu.SemaphoreType.DMA,
        ],
    )(_sc_degree_body)


def _tc_lookup_body(x_ref, deg_ref, win_ref, wout_ref, o_ref):
    part = deg_ref[...]
    din = jnp.minimum(part[0:1, :] + part[2:3, :], MAX_DEG - 1)
    dout = jnp.minimum(part[1:2, :] + part[3:4, :], MAX_DEG - 1)
    iot = lax.broadcasted_iota(jnp.int32, (MAX_DEG, NODE_BLK), 0)
    oh_in = (jnp.broadcast_to(din, (MAX_DEG, NODE_BLK)) == iot)
    oh_out = (jnp.broadcast_to(dout, (MAX_DEG, NODE_BLK)) == iot)
    dn = (((0,), (0,)), ((), ()))
    acc = lax.dot_general(oh_in.astype(jnp.float32), win_ref[...], dn,
                          precision=lax.Precision.HIGHEST,
                          preferred_element_type=jnp.float32)
    acc = acc + lax.dot_general(oh_out.astype(jnp.float32), wout_ref[...], dn,
                                precision=lax.Precision.HIGHEST,
                                preferred_element_type=jnp.float32)
    o_ref[...] = x_ref[...] + acc


def _tc_lookup(x, partial, w_in, w_out):
    return pl.pallas_call(
        _tc_lookup_body,
        grid=(TC_GRID,),
        in_specs=[
            pl.BlockSpec((NODE_BLK, EMB_DIM), lambda i: (i, _i(0))),
            pl.BlockSpec((8, NODE_BLK), lambda i: (_i(0), i)),
            pl.BlockSpec((MAX_DEG, EMB_DIM), lambda i: (_i(0), _i(0))),
            pl.BlockSpec((MAX_DEG, EMB_DIM), lambda i: (_i(0), _i(0))),
        ],
        out_specs=pl.BlockSpec((NODE_BLK, EMB_DIM), lambda i: (i, _i(0))),
        out_shape=jax.ShapeDtypeStruct((N_NODES, EMB_DIM), jnp.float32),
    )(x, partial, w_in, w_out)


def kernel(x, edge_index, in_emb_weight, out_emb_weight):
    e32 = edge_index.astype(jnp.int32)
    partial = _get_sc_degree()(e32[0], e32[1])
    return _tc_lookup(x, partial, in_emb_weight, out_emb_weight)
